# Initial kernel scaffold; baseline (speedup 1.0000x reference)
import jax
import jax.numpy as jnp
from jax.experimental import pallas as pl


def kernel(embeddings, position_ids, table):
    raise NotImplementedError("write your pallas kernel here")


# speedup vs baseline: 1.0249x; 1.0249x over previous
import functools

import jax
import jax.numpy as jnp
from jax import lax
from jax.experimental import pallas as pl
from jax.experimental.pallas import tpu as pltpu
from jax.experimental.pallas import tpu_sc as plsc

D_MODEL = 1024
MAX_LEN = 4096
N_ROWS = 16384

NUM_CORES = 2
NUM_SUBCORES = 16
NW = NUM_CORES * NUM_SUBCORES
R = N_ROWS // NW
K = 32


def _body(emb_hbm, pos_hbm, table_hbm, out_hbm, idx_v, emb_v, rows_v, sem):
    wid = lax.axis_index("s") * NUM_CORES + lax.axis_index("c")
    base = pl.multiple_of(wid * R, R)

    pltpu.sync_copy(pos_hbm.at[pl.ds(base, R)], idx_v)
    for i in range(R // 16):
        sl = pl.ds(i * 16, 16)
        idx_v[sl] = jnp.clip(idx_v[sl], 0, MAX_LEN - 1)

    def chunk(c, carry):
        cb = pl.multiple_of(c * K, K)
        rows = pl.ds(base + cb, K)
        gather = pltpu.async_copy(
            table_hbm.at[idx_v.at[pl.ds(cb, K)]], rows_v, sem
        )
        pltpu.sync_copy(emb_hbm.at[rows], emb_v)
        gather.wait()

        def add_row(j, carry2):
            for t in range(D_MODEL // 16):
                sl = pl.ds(t * 16, 16)
                emb_v[j, sl] = emb_v[j, sl] + rows_v[j, sl]
            return carry2

        lax.fori_loop(0, K, add_row, 0)
        pltpu.sync_copy(emb_v, out_hbm.at[rows])
        return carry

    lax.fori_loop(0, R // K, chunk, 0)


@jax.jit
def _lookup_add(emb2, pos, table):
    mesh = plsc.VectorSubcoreMesh(core_axis_name="c", subcore_axis_name="s")
    return pl.kernel(
        _body,
        out_type=jax.ShapeDtypeStruct((N_ROWS, D_MODEL), jnp.float32),
        mesh=mesh,
        scratch_types=[
            pltpu.VMEM((R,), jnp.int32),
            pltpu.VMEM((K, D_MODEL), jnp.float32),
            pltpu.VMEM((K, D_MODEL), jnp.float32),
            pltpu.SemaphoreType.DMA,
        ],
    )(emb2, pos, table)


def kernel(embeddings, position_ids, table):
    B, S, D = embeddings.shape
    emb2 = embeddings.reshape(B * S, D)
    pos = position_ids.reshape(B * S).astype(jnp.int32)
    out = _lookup_add(emb2, pos, table)
    return out.reshape(B, S, D)


# speedup vs baseline: 1.5144x; 1.4776x over previous
import jax
import jax.numpy as jnp
from jax import lax
from jax.experimental import pallas as pl
from jax.experimental.pallas import tpu as pltpu
from jax.experimental.pallas import tpu_sc as plsc

D_MODEL = 1024
MAX_LEN = 4096
N_ROWS = 16384

NUM_CORES = 2
NUM_SUBCORES = 16
NW = NUM_CORES * NUM_SUBCORES
R = N_ROWS // NW
K = 16
NCHUNKS = R // K
NBUF = 2


def _body(emb_hbm, pos_hbm, table_hbm, out_hbm, idx_v,
          emb0, emb1, rows0, rows1, out0, out1,
          gsem0, gsem1, esem0, esem1, osem0, osem1):
    embs = (emb0, emb1)
    rowss = (rows0, rows1)
    outs = (out0, out1)
    gsems = (gsem0, gsem1)
    esems = (esem0, esem1)
    osems = (osem0, osem1)

    wid = lax.axis_index("s") * NUM_CORES + lax.axis_index("c")
    base = pl.multiple_of(wid * R, R)

    pltpu.sync_copy(pos_hbm.at[pl.ds(base, R)], idx_v)
    for i in range(R // 16):
        sl = pl.ds(i * 16, 16)
        idx_v[sl] = jnp.clip(idx_v[sl], 0, MAX_LEN - 1)

    def start_in(c, b):
        cb = c * K
        g = pltpu.async_copy(
            table_hbm.at[idx_v.at[pl.ds(cb, K)]], rowss[b], gsems[b])
        e = pltpu.async_copy(
            emb_hbm.at[pl.ds(base + cb, K)], embs[b], esems[b])
        return g, e

    in_descs = {}
    out_descs = {}
    for b in range(NBUF):
        in_descs[b] = start_in(b, b)

    for c in range(NCHUNKS):
        b = c % NBUF
        g, e = in_descs[b]
        g.wait()
        e.wait()
        if c >= NBUF:
            out_descs[c - NBUF].wait()

        def add_row(j, carry):
            for t in range(D_MODEL // 16):
                sl = pl.ds(t * 16, 16)
                outs[b][j, sl] = embs[b][j, sl] + rowss[b][j, sl]
            return carry

        lax.fori_loop(0, K, add_row, 0)

        out_descs[c] = pltpu.async_copy(
            outs[b], out_hbm.at[pl.ds(base + c * K, K)], osems[b])
        if c + NBUF < NCHUNKS:
            in_descs[b] = start_in(c + NBUF, b)

    for c in range(NCHUNKS - NBUF, NCHUNKS):
        out_descs[c].wait()


@jax.jit
def _lookup_add(emb2, pos, table):
    mesh = plsc.VectorSubcoreMesh(core_axis_name="c", subcore_axis_name="s")
    return pl.kernel(
        _body,
        out_type=jax.ShapeDtypeStruct((N_ROWS, D_MODEL), jnp.float32),
        mesh=mesh,
        scratch_types=[
            pltpu.VMEM((R,), jnp.int32),
            pltpu.VMEM((K, D_MODEL), jnp.float32),
            pltpu.VMEM((K, D_MODEL), jnp.float32),
            pltpu.VMEM((K, D_MODEL), jnp.float32),
            pltpu.VMEM((K, D_MODEL), jnp.float32),
            pltpu.VMEM((K, D_MODEL), jnp.float32),
            pltpu.VMEM((K, D_MODEL), jnp.float32),
            pltpu.SemaphoreType.DMA,
            pltpu.SemaphoreType.DMA,
            pltpu.SemaphoreType.DMA,
            pltpu.SemaphoreType.DMA,
            pltpu.SemaphoreType.DMA,
            pltpu.SemaphoreType.DMA,
        ],
    )(emb2, pos, table)


def kernel(embeddings, position_ids, table):
    B, S, D = embeddings.shape
    emb2 = embeddings.reshape(B * S, D)
    pos = position_ids.reshape(B * S).astype(jnp.int32)
    out = _lookup_add(emb2, pos, table)
    return out.reshape(B, S, D)
